# baseline (speedup 1.0000x reference)
import functools
import math

import jax
import jax.numpy as jnp
from jax import lax
from jax.experimental import pallas as pl
from jax.experimental.pallas import tpu as pltpu
from jax.experimental.pallas import tpu_sc as plsc

GX, GY, Z = 512, 512, 256
NC, NS = 2, 16
NW = NC * NS
ROWS = GX * GY
RPW = ROWS // NW
CH = 64
NCH = RPW // CH
NBUF = 4

_mesh = plsc.VectorSubcoreMesh(core_axis_name="c", subcore_axis_name="s")


def kernel(x, t, W, gx, gy):
    wf = W.reshape(ROWS, Z)

    @functools.partial(
        pl.kernel,
        mesh=_mesh,
        out_type=jax.ShapeDtypeStruct((GX, GY), jnp.float32),
        scratch_types=[
            pltpu.VMEM((NBUF, CH, Z), jnp.float32),
        ] + [pltpu.SemaphoreType.DMA] * NBUF,
    )
    def sc_run(w_hbm, out_hbm, buf, *sems):
        c = lax.axis_index("c")
        s = lax.axis_index("s")
        wid = s * NC + c
        base = wid * RPW

        for b in range(NBUF):
            pltpu.make_async_copy(
                w_hbm.at[pl.ds(base + b * CH, CH)], buf.at[b], sems[b]
            ).start()

        def step(g, carry):
            for b in range(NBUF):
                ch = NBUF * g + b
                pltpu.make_async_copy(
                    w_hbm.at[pl.ds(base + ch * CH, CH)], buf.at[b], sems[b]
                ).wait()

                @pl.when(ch + NBUF < NCH)
                def _():
                    pltpu.make_async_copy(
                        w_hbm.at[pl.ds(base + (ch + NBUF) * CH, CH)],
                        buf.at[b],
                        sems[b],
                    ).start()

            return carry

        lax.fori_loop(0, NCH // NBUF, step, jnp.int32(0))

    return sc_run(wf)
